# baseline (speedup 1.0000x reference)
import functools

import jax
import jax.numpy as jnp
from jax import lax
from jax.experimental import pallas as pl
from jax.experimental.pallas import tpu as pltpu
from jax.experimental.pallas import tpu_sc as plsc

_C = 16


def kernel(x, pos_embedding):
    batch, seq_len = x.shape
    max_len, d_model = pos_embedding.shape

    info = plsc.get_sparse_core_info()
    nc, ns = info.num_cores, info.num_subcores
    nw = nc * ns
    rows_per_w = seq_len // nw
    k = rows_per_w // _C

    mesh = plsc.VectorSubcoreMesh(core_axis_name="c", subcore_axis_name="s")

    @functools.partial(
        pl.kernel,
        mesh=mesh,
        out_type=jax.ShapeDtypeStruct((batch, seq_len, d_model), jnp.float32),
        scratch_types=[
            pltpu.VMEM((2, _C, d_model), jnp.float32),
            pltpu.SemaphoreType.DMA((2,)),
            pltpu.SemaphoreType.DMA((2,)),
        ],
    )
    def k_sc(table_hbm, out_hbm, buf, rsem, wsem):
        wid = lax.axis_index("s") * nc + lax.axis_index("c")
        base = wid * rows_per_w

        reads = []
        writes = {}
        for j in range(k):
            s = j % 2
            r0 = base + j * _C
            if j >= 2:
                for c in writes.pop(j - 2):
                    c.wait()
            rd = pltpu.make_async_copy(
                table_hbm.at[pl.ds(r0, _C)], buf.at[s], rsem.at[s]
            )
            rd.start()
            rd.wait()
            ws = []
            for b in range(batch):
                wr = pltpu.make_async_copy(
                    buf.at[s], out_hbm.at[b, pl.ds(r0, _C)], wsem.at[s]
                )
                wr.start()
                ws.append(wr)
            writes[j] = ws
        for js in sorted(writes):
            for c in writes[js]:
                c.wait()

    return k_sc(pos_embedding)
